# baseline (speedup 1.0000x reference)
import functools
import math

import jax
import jax.numpy as jnp
from jax import lax
from jax.experimental import pallas as pl
from jax.experimental.pallas import tpu as pltpu
from jax.experimental.pallas import tpu_sc as plsc

_SIZE = 32000
_PAD = 0
_SMOOTH = 0.1
_CONF = 1.0 - _SMOOTH
_EPS = _SMOOTH / (_SIZE - 1)
_C = (_SIZE - 1) * _EPS * math.log(_EPS) + _CONF * math.log(_CONF)

_N = 4096
_R = 128
_CB = 32000

_NC = 2
_NS = 16
_L = 16
_NW = _NC * _NS

_RPT = 48
_ROWS_SC = _NW * _RPT
_ROWS_TC = _N - _ROWS_SC
_U = 16


def _sc_body(tgt_hbm, x_hbm, loss_hbm, tok_hbm, tgt_v, buf0, buf1, outv, sem0, sem1):
    wid = lax.axis_index("s") * _NC + lax.axis_index("c")
    row0 = _ROWS_TC + wid * _RPT
    pltpu.sync_copy(tgt_hbm.at[pl.ds(row0, _RPT)], tgt_v)
    bufs = (buf0, buf1)
    sems = (sem0, sem1)
    handles = [None] * _RPT
    handles[0] = pltpu.async_copy(x_hbm.at[row0], buf0, sem0)

    iota = lax.iota(jnp.int32, _L)
    lane0f = jnp.where(iota == 0, 1.0, 0.0).astype(jnp.float32)
    iotas = [iota + u * _L for u in range(_U)]
    acc_row = jnp.zeros((_L,), jnp.float32)
    acc_xt = jnp.zeros((_L,), jnp.float32)
    acc_tok = jnp.zeros((_L,), jnp.float32)
    zeros = tuple(jnp.zeros((_L,), jnp.float32) for _ in range(2 * _U))

    for k in range(_RPT):
        if k + 1 < _RPT:
            handles[k + 1] = pltpu.async_copy(
                x_hbm.at[row0 + k + 1], bufs[(k + 1) % 2], sems[(k + 1) % 2]
            )
        handles[k].wait()
        buf = bufs[k % 2]

        chunk = tgt_v[pl.ds((k // _L) * _L, _L)]
        tb = chunk.at[jnp.full((_L,), k % _L, jnp.int32)].get(
            mode="promise_in_bounds"
        )
        maskf = jnp.where(tb != _PAD, 1.0, 0.0).astype(jnp.float32)

        def body(c, carry, _buf=buf, _tb=tb):
            b = c * (_L * _U)
            out = []
            for u in range(_U):
                v = _buf[pl.ds(b + u * _L, _L)]
                out.append(carry[u] + v)
            for u in range(_U):
                v = _buf[pl.ds(b + u * _L, _L)]
                out.append(
                    carry[_U + u] + jnp.where(iotas[u] + b == _tb, v, 0.0)
                )
            return tuple(out)

        carry = lax.fori_loop(0, _SIZE // (_L * _U), body, zeros)
        rs = carry[0]
        ht = carry[_U]
        for u in range(1, _U):
            rs = rs + carry[u]
            ht = ht + carry[_U + u]
        acc_row = acc_row + rs * maskf
        acc_xt = acc_xt + ht * maskf
        acc_tok = acc_tok + maskf * lane0f

    loss16 = (
        _C * acc_tok - _EPS * acc_row - (_CONF - _EPS) * acc_xt
    )
    outv[...] = loss16
    pltpu.sync_copy(outv, loss_hbm.at[wid])
    outv[...] = acc_tok
    pltpu.sync_copy(outv, tok_hbm.at[wid])


_sc_call = functools.partial(
    pl.kernel,
    out_type=(
        jax.ShapeDtypeStruct((_NW, _L), jnp.float32),
        jax.ShapeDtypeStruct((_NW, _L), jnp.float32),
    ),
    mesh=plsc.VectorSubcoreMesh(core_axis_name="c", subcore_axis_name="s"),
    scratch_types=[
        pltpu.VMEM((_RPT,), jnp.int32),
        pltpu.VMEM((_SIZE,), jnp.float32),
        pltpu.VMEM((_SIZE,), jnp.float32),
        pltpu.VMEM((_L,), jnp.float32),
        pltpu.SemaphoreType.DMA,
        pltpu.SemaphoreType.DMA,
    ],
)(_sc_body)


def _tc_kernel(tgt_ref, x_ref, loss_ref, tok_ref, acc_ref, cnt_ref):
    i = pl.program_id(0)
    ni = pl.num_programs(0)

    @pl.when(i == 0)
    def _init():
        acc_ref[0, 0] = 0.0
        cnt_ref[0, 0] = 0.0

    x = x_ref[...]
    tgt = tgt_ref[0]
    tgt_col = tgt.reshape(_R, 1)
    maskv = tgt_col != _PAD

    rowsum = jnp.sum(x, axis=1, keepdims=True)
    col = jax.lax.broadcasted_iota(jnp.int32, (_R, _CB), 1)
    xt = jnp.sum(jnp.where(col == tgt_col, x, 0.0), axis=1, keepdims=True)
    contrib = jnp.where(maskv, -_EPS * rowsum - (_CONF - _EPS) * xt, 0.0)
    mask_cnt = jnp.sum(maskv.astype(jnp.float32))
    acc_ref[0, 0] += jnp.sum(contrib) + _C * mask_cnt
    cnt_ref[0, 0] += mask_cnt

    @pl.when(i == ni - 1)
    def _finish():
        loss_ref[0, 0] = acc_ref[0, 0]
        tok_ref[0, 0] = cnt_ref[0, 0]


def kernel(x, target):
    g = _ROWS_TC // _R
    tgt32 = target.astype(jnp.int32)
    sc_loss, sc_tok = _sc_call(tgt32, x)
    tgt_blocks = tgt32.reshape(_N // _R, 1, _R)
    tc_loss, tc_tok = pl.pallas_call(
        _tc_kernel,
        grid=(g,),
        in_specs=[
            pl.BlockSpec((1, 1, _R), lambda i: (i, 0, 0)),
            pl.BlockSpec((_R, _CB), lambda i: (i, 0)),
        ],
        out_specs=[
            pl.BlockSpec(memory_space=pltpu.SMEM),
            pl.BlockSpec(memory_space=pltpu.SMEM),
        ],
        out_shape=[
            jax.ShapeDtypeStruct((1, 1), jnp.float32),
            jax.ShapeDtypeStruct((1, 1), jnp.float32),
        ],
        scratch_shapes=[
            pltpu.SMEM((1, 1), jnp.float32),
            pltpu.SMEM((1, 1), jnp.float32),
        ],
    )(tgt_blocks, x)
    num = tc_loss[0, 0] + jnp.sum(sc_loss)
    den = tc_tok[0, 0] + jnp.sum(sc_tok)
    return num / den


# speedup vs baseline: 1.1739x; 1.1739x over previous
import math

import jax
import jax.numpy as jnp
from jax.experimental import pallas as pl
from jax.experimental.pallas import tpu as pltpu

_SIZE = 32000
_PAD = 0
_SMOOTH = 0.1
_CONF = 1.0 - _SMOOTH
_EPS = _SMOOTH / (_SIZE - 1)
_C = (_SIZE - 1) * _EPS * math.log(_EPS) + _CONF * math.log(_CONF)

_R = 128
_CB = 32000


def _ls_kernel(tgt_ref, x_ref, out_ref, acc_ref, tok_ref):
    i = pl.program_id(0)
    ni = pl.num_programs(0)

    @pl.when(i == 0)
    def _init():
        acc_ref[0, 0] = 0.0
        tok_ref[0, 0] = 0.0

    x = x_ref[...]
    tgt = tgt_ref[0]
    tgt_col = tgt.reshape(_R, 1)
    maskv = tgt_col != _PAD

    rowsum = jnp.sum(x, axis=1, keepdims=True)
    col = jax.lax.broadcasted_iota(jnp.int32, (_R, _CB), 1)
    xt = jnp.sum(jnp.where(col == tgt_col, x, 0.0), axis=1, keepdims=True)
    contrib = jnp.where(maskv, -_EPS * rowsum - (_CONF - _EPS) * xt, 0.0)
    mask_cnt = jnp.sum(maskv.astype(jnp.float32))
    acc_ref[0, 0] += jnp.sum(contrib) + _C * mask_cnt
    tok_ref[0, 0] += mask_cnt

    @pl.when(i == ni - 1)
    def _finish():
        out_ref[0, 0] = acc_ref[0, 0] / tok_ref[0, 0]


def kernel(x, target):
    n = x.shape[0]
    g = n // _R
    tgt_blocks = target.astype(jnp.int32).reshape(g, 1, _R)
    out = pl.pallas_call(
        _ls_kernel,
        grid=(g,),
        in_specs=[
            pl.BlockSpec((1, 1, _R), lambda i: (i, 0, 0)),
            pl.BlockSpec((_R, _CB), lambda i: (i, 0)),
        ],
        out_specs=pl.BlockSpec(memory_space=pltpu.SMEM),
        out_shape=jax.ShapeDtypeStruct((1, 1), jnp.float32),
        scratch_shapes=[
            pltpu.SMEM((1, 1), jnp.float32),
            pltpu.SMEM((1, 1), jnp.float32),
        ],
    )(tgt_blocks, x)
    return out[0, 0]
